# baseline (speedup 1.0000x reference)
import functools

import jax
import jax.numpy as jnp
from jax import lax
from jax.experimental import pallas as pl
from jax.experimental.pallas import tpu as pltpu
from jax.experimental.pallas import tpu_sc as plsc

EMB = 16
HALF = 8
BATCH = 16384
TBL = 1000000
NC = 2
NS = 16
NW = NC * NS
BPW = BATCH // NW
CH = 128
L = 16
NG = BPW // L

TCOL = 262144
NBLK = (TBL + TCOL - 1) // TCOL
BLKW = 8 * TCOL
PLANE = NBLK * BLKW


def _repack_body(t_ref, out_ref):
    x = t_ref[...]
    out_ref[...] = x.reshape(8, TCOL // 128, 128).transpose(1, 0, 2).reshape(BLKW)


def _tc_repack(t, p):
    grid = (NBLK,)
    return pl.pallas_call(
        _repack_body,
        grid=grid,
        in_specs=[pl.BlockSpec((8, TCOL), lambda m: (p, m))],
        out_specs=pl.BlockSpec((BLKW,), lambda m: (m,)),
        out_shape=jax.ShapeDtypeStruct((PLANE,), jnp.float32),
    )(t)


def _gather_body(idx_hbm, tab_hbm, out_hbm, idx_v, idxf, rows, sem):
    wid = lax.axis_index("s") * NC + lax.axis_index("c")
    base = wid * BPW
    pltpu.sync_copy(idx_hbm.at[pl.ds(base, BPW)], idx_v)
    for g in range(NG):
        r = idx_v[pl.ds(g * L, L)]
        t = ((r >> 7) << 10) + (r & 127)
        for c in range(HALF):
            idxf[c, pl.ds(g * L, L)] = t + c * 128
    copies = []
    for c in range(HALF):
        for ch in range(BPW // CH):
            off = ch * CH
            copies.append(pltpu.async_copy(
                tab_hbm.at[idxf.at[c, pl.ds(off, CH)]],
                rows.at[c, pl.ds(off, CH)], sem))
    for cp in copies:
        cp.wait()
    pltpu.sync_copy(rows, out_hbm.at[:, pl.ds(base, BPW)])


def _sc_gather(idx, flat):
    mesh = plsc.VectorSubcoreMesh(core_axis_name="c", subcore_axis_name="s")
    f = functools.partial(
        pl.kernel,
        mesh=mesh,
        out_type=jax.ShapeDtypeStruct((HALF, BATCH), jnp.float32),
        scratch_types=[
            pltpu.VMEM((BPW,), jnp.int32),
            pltpu.VMEM((HALF, BPW), jnp.int32),
            pltpu.VMEM((HALF, BPW), jnp.float32),
            pltpu.SemaphoreType.DMA,
        ],
        compiler_params=pltpu.CompilerParams(use_tc_tiling_on_sc=False),
    )(_gather_body)
    return f(idx, flat)


def _mlp_body(x0_ref, x1_ref, x2_ref, x3_ref, w0_ref, w1_ref, w2_ref, w3_ref,
              b1_ref, w2c_ref, b2_ref, out_ref):
    h = jnp.dot(w0_ref[...], x0_ref[...], preferred_element_type=jnp.float32)
    h = h + jnp.dot(w1_ref[...], x1_ref[...], preferred_element_type=jnp.float32)
    h = h + jnp.dot(w2_ref[...], x2_ref[...], preferred_element_type=jnp.float32)
    h = h + jnp.dot(w3_ref[...], x3_ref[...], preferred_element_type=jnp.float32)
    h = jnp.maximum(h + b1_ref[...], 0.0)
    o = jnp.sum(h * w2c_ref[...], axis=0, keepdims=True) + b2_ref[...]
    out_ref[...] = 1.0 / (1.0 + jnp.exp(-o))


BN = 4096


def _tc_mlp(xs, ws, b1c, w2c, b2c):
    grid = (BATCH // BN,)
    x_spec = pl.BlockSpec((HALF, BN), lambda m: (0, m))
    w_spec = pl.BlockSpec((EMB, HALF), lambda m: (0, 0))
    return pl.pallas_call(
        _mlp_body,
        grid=grid,
        in_specs=[x_spec] * 4 + [w_spec] * 4 + [
            pl.BlockSpec((EMB, 1), lambda m: (0, 0)),
            pl.BlockSpec((EMB, 1), lambda m: (0, 0)),
            pl.BlockSpec((1, 1), lambda m: (0, 0)),
        ],
        out_specs=pl.BlockSpec((1, BN), lambda m: (0, m)),
        out_shape=jax.ShapeDtypeStruct((1, BATCH), jnp.float32),
    )(*xs, *ws, b1c, w2c, b2c)


def kernel(u, i, user_emb, item_emb, W1, b1, W2, b2):
    u = u.astype(jnp.int32)
    i = i.astype(jnp.int32)
    ut = user_emb.T
    it = item_emb.T
    uf0 = _tc_repack(ut, 0)
    ue0 = _sc_gather(u, uf0)
    uf1 = _tc_repack(ut, 1)
    ue1 = _sc_gather(u, uf1)
    if0 = _tc_repack(it, 0)
    ie0 = _sc_gather(i, if0)
    if1 = _tc_repack(it, 1)
    ie1 = _sc_gather(i, if1)
    ws = [W1[0:8].T, W1[8:16].T, W1[16:24].T, W1[24:32].T]
    b1c = b1.reshape(EMB, 1)
    w2c = W2.reshape(EMB, 1)
    b2c = b2.reshape(1, 1)
    out_t = _tc_mlp([ue0, ue1, ie0, ie1], ws, b1c, w2c, b2c)
    return out_t.reshape(BATCH, 1)


# speedup vs baseline: 1.0161x; 1.0161x over previous
import functools

import jax
import jax.numpy as jnp
from jax import lax
from jax.experimental import pallas as pl
from jax.experimental.pallas import tpu as pltpu
from jax.experimental.pallas import tpu_sc as plsc

EMB = 16
BATCH = 16384
TBL = 1000000
NC = 2
NS = 16
NW = NC * NS
BPW = BATCH // NW
CH = 128
L = 16
NG = BPW // L

TCOL = 262144
NBLK = (TBL + TCOL - 1) // TCOL
BLKW = 8 * TCOL
PLANE = NBLK * BLKW


def _repack_body(t_ref, out_ref):
    x = t_ref[...]
    out_ref[...] = x.reshape(8, TCOL // 128, 128).transpose(1, 0, 2).reshape(BLKW)


def _tc_repack(t):
    grid = (2, NBLK)
    return pl.pallas_call(
        _repack_body,
        grid=grid,
        in_specs=[pl.BlockSpec((8, TCOL), lambda p, m: (p, m))],
        out_specs=pl.BlockSpec((BLKW,), lambda p, m: (p * NBLK + m,)),
        out_shape=jax.ShapeDtypeStruct((2 * PLANE,), jnp.float32),
    )(t)


def _gather_body(idx_hbm, tab_hbm, out_hbm, idx_v, idxf, rows, sem):
    wid = lax.axis_index("s") * NC + lax.axis_index("c")
    base = wid * BPW
    pltpu.sync_copy(idx_hbm.at[pl.ds(base, BPW)], idx_v)
    for g in range(NG):
        r = idx_v[pl.ds(g * L, L)]
        t = ((r >> 7) << 10) + (r & 127)
        for c in range(EMB):
            off = (c // 8) * PLANE + (c % 8) * 128
            idxf[c, pl.ds(g * L, L)] = t + off
    copies = []
    for c in range(EMB):
        for ch in range(BPW // CH):
            off = ch * CH
            copies.append(pltpu.async_copy(
                tab_hbm.at[idxf.at[c, pl.ds(off, CH)]],
                rows.at[c, pl.ds(off, CH)], sem))
    for cp in copies:
        cp.wait()
    pltpu.sync_copy(rows, out_hbm.at[:, pl.ds(base, BPW)])


def _sc_gather(idx, flat):
    mesh = plsc.VectorSubcoreMesh(core_axis_name="c", subcore_axis_name="s")
    f = functools.partial(
        pl.kernel,
        mesh=mesh,
        out_type=jax.ShapeDtypeStruct((EMB, BATCH), jnp.float32),
        scratch_types=[
            pltpu.VMEM((BPW,), jnp.int32),
            pltpu.VMEM((EMB, BPW), jnp.int32),
            pltpu.VMEM((EMB, BPW), jnp.float32),
            pltpu.SemaphoreType.DMA,
        ],
        compiler_params=pltpu.CompilerParams(use_tc_tiling_on_sc=False),
    )(_gather_body)
    return f(idx, flat)


def _mlp_body(ue_ref, ie_ref, w1a_ref, w1b_ref, b1_ref, w2_ref, b2_ref, out_ref):
    h = jnp.dot(w1a_ref[...], ue_ref[...], preferred_element_type=jnp.float32)
    h = h + jnp.dot(w1b_ref[...], ie_ref[...], preferred_element_type=jnp.float32)
    h = jnp.maximum(h + b1_ref[...], 0.0)
    o = jnp.sum(h * w2_ref[...], axis=0, keepdims=True) + b2_ref[...]
    out_ref[...] = 1.0 / (1.0 + jnp.exp(-o))


BN = 4096


def _tc_mlp(ue_t, ie_t, w1a_t, w1b_t, b1c, w2c, b2c):
    grid = (BATCH // BN,)
    return pl.pallas_call(
        _mlp_body,
        grid=grid,
        in_specs=[
            pl.BlockSpec((EMB, BN), lambda m: (0, m)),
            pl.BlockSpec((EMB, BN), lambda m: (0, m)),
            pl.BlockSpec((EMB, EMB), lambda m: (0, 0)),
            pl.BlockSpec((EMB, EMB), lambda m: (0, 0)),
            pl.BlockSpec((EMB, 1), lambda m: (0, 0)),
            pl.BlockSpec((EMB, 1), lambda m: (0, 0)),
            pl.BlockSpec((1, 1), lambda m: (0, 0)),
        ],
        out_specs=pl.BlockSpec((1, BN), lambda m: (0, m)),
        out_shape=jax.ShapeDtypeStruct((1, BATCH), jnp.float32),
    )(ue_t, ie_t, w1a_t, w1b_t, b1c, w2c, b2c)


def kernel(u, i, user_emb, item_emb, W1, b1, W2, b2):
    u = u.astype(jnp.int32)
    i = i.astype(jnp.int32)
    uf = _tc_repack(user_emb.T)
    ue_t = _sc_gather(u, uf)
    if_ = _tc_repack(item_emb.T)
    ie_t = _sc_gather(i, if_)
    w1a_t = W1[:EMB].T
    w1b_t = W1[EMB:].T
    b1c = b1.reshape(EMB, 1)
    w2c = W2.reshape(EMB, 1)
    b2c = b2.reshape(1, 1)
    out_t = _tc_mlp(ue_t, ie_t, w1a_t, w1b_t, b1c, w2c, b2c)
    return out_t.reshape(BATCH, 1)


# speedup vs baseline: 1.0227x; 1.0066x over previous
import functools

import jax
import jax.numpy as jnp
from jax import lax
from jax.experimental import pallas as pl
from jax.experimental.pallas import tpu as pltpu
from jax.experimental.pallas import tpu_sc as plsc

EMB = 16
BATCH = 16384
TBL = 1000000
NC = 2
NS = 16
NW = NC * NS
BPW = BATCH // NW
CH = 128
L = 16
NG = BPW // L

TCOL = 333952
NBLK = (TBL + TCOL - 1) // TCOL
BLKW = 8 * TCOL
PLANE = NBLK * BLKW


def _repack_body(t_ref, out_ref):
    x = t_ref[...]
    out_ref[...] = x.reshape(8, TCOL // 128, 128).transpose(1, 0, 2).reshape(BLKW)


def _tc_repack(t):
    grid = (2, NBLK)
    return pl.pallas_call(
        _repack_body,
        grid=grid,
        in_specs=[pl.BlockSpec((8, TCOL), lambda p, m: (p, m))],
        out_specs=pl.BlockSpec((BLKW,), lambda p, m: (p * NBLK + m,)),
        out_shape=jax.ShapeDtypeStruct((2 * PLANE,), jnp.float32),
    )(t)


def _gather_body(idx_hbm, tab_hbm, out_hbm, idx_v, idxf, rows, sem):
    wid = lax.axis_index("s") * NC + lax.axis_index("c")
    base = wid * BPW
    pltpu.sync_copy(idx_hbm.at[pl.ds(base, BPW)], idx_v)
    for g in range(NG):
        r = idx_v[pl.ds(g * L, L)]
        t = ((r >> 7) << 10) + (r & 127)
        for c in range(EMB):
            off = (c // 8) * PLANE + (c % 8) * 128
            idxf[c, pl.ds(g * L, L)] = t + off
    copies = []
    for c in range(EMB):
        for ch in range(BPW // CH):
            off = ch * CH
            copies.append(pltpu.async_copy(
                tab_hbm.at[idxf.at[c, pl.ds(off, CH)]],
                rows.at[c, pl.ds(off, CH)], sem))
    for cp in copies:
        cp.wait()
    pltpu.sync_copy(rows, out_hbm.at[:, pl.ds(base, BPW)])


def _sc_gather(idx, flat):
    mesh = plsc.VectorSubcoreMesh(core_axis_name="c", subcore_axis_name="s")
    f = functools.partial(
        pl.kernel,
        mesh=mesh,
        out_type=jax.ShapeDtypeStruct((EMB, BATCH), jnp.float32),
        scratch_types=[
            pltpu.VMEM((BPW,), jnp.int32),
            pltpu.VMEM((EMB, BPW), jnp.int32),
            pltpu.VMEM((EMB, BPW), jnp.float32),
            pltpu.SemaphoreType.DMA,
        ],
        compiler_params=pltpu.CompilerParams(use_tc_tiling_on_sc=False),
    )(_gather_body)
    return f(idx, flat)


def _mlp_body(ue_ref, ie_ref, w1a_ref, w1b_ref, b1_ref, w2_ref, b2_ref, out_ref):
    h = jnp.dot(w1a_ref[...], ue_ref[...], preferred_element_type=jnp.float32)
    h = h + jnp.dot(w1b_ref[...], ie_ref[...], preferred_element_type=jnp.float32)
    h = jnp.maximum(h + b1_ref[...], 0.0)
    o = jnp.sum(h * w2_ref[...], axis=0, keepdims=True) + b2_ref[...]
    out_ref[...] = 1.0 / (1.0 + jnp.exp(-o))


BN = 4096


def _tc_mlp(ue_t, ie_t, w1a_t, w1b_t, b1c, w2c, b2c):
    grid = (BATCH // BN,)
    return pl.pallas_call(
        _mlp_body,
        grid=grid,
        in_specs=[
            pl.BlockSpec((EMB, BN), lambda m: (0, m)),
            pl.BlockSpec((EMB, BN), lambda m: (0, m)),
            pl.BlockSpec((EMB, EMB), lambda m: (0, 0)),
            pl.BlockSpec((EMB, EMB), lambda m: (0, 0)),
            pl.BlockSpec((EMB, 1), lambda m: (0, 0)),
            pl.BlockSpec((EMB, 1), lambda m: (0, 0)),
            pl.BlockSpec((1, 1), lambda m: (0, 0)),
        ],
        out_specs=pl.BlockSpec((1, BN), lambda m: (0, m)),
        out_shape=jax.ShapeDtypeStruct((1, BATCH), jnp.float32),
    )(ue_t, ie_t, w1a_t, w1b_t, b1c, w2c, b2c)


def kernel(u, i, user_emb, item_emb, W1, b1, W2, b2):
    u = u.astype(jnp.int32)
    i = i.astype(jnp.int32)
    uf = _tc_repack(user_emb.T)
    ue_t = _sc_gather(u, uf)
    if_ = _tc_repack(item_emb.T)
    ie_t = _sc_gather(i, if_)
    w1a_t = W1[:EMB].T
    w1b_t = W1[EMB:].T
    b1c = b1.reshape(EMB, 1)
    w2c = W2.reshape(EMB, 1)
    b2c = b2.reshape(1, 1)
    out_t = _tc_mlp(ue_t, ie_t, w1a_t, w1b_t, b1c, w2c, b2c)
    return out_t.reshape(BATCH, 1)
